# Initial kernel scaffold; baseline (speedup 1.0000x reference)
import jax
import jax.numpy as jnp
from jax.experimental import pallas as pl


def kernel(A0_indices, A0_values, A1_indices, A1_values, X, W):
    raise NotImplementedError("write your pallas kernel here")


# speedup vs baseline: 1.4865x; 1.4865x over previous
import functools

import jax
import jax.numpy as jnp
from jax import lax
from jax.experimental import pallas as pl
from jax.experimental.pallas import tpu as pltpu
from jax.experimental.pallas import tpu_sc as plsc

N = 10000
E = 160000
D = 128
DH = 64
NT = 16
CHUNK = 80
NCH = E // NT // CHUNK
NCHP = 128
ZB = 80
ZSTRIDE = 640
VL = 16


def _sc_body(t_a, t_b, c0, r0, v0, c1, r1, v1,
             y1a_a, y1a_b, y2a_a, y2a_b, y1b_a, y1b_b, y2b_a, y2b_b,
             col_v, row_v, val_v, rows_v, zbuf, bounce, acc, sem):
    cid = lax.axis_index("c")
    sid = lax.axis_index("s")
    coff = cid * N
    zbase = sid * ZSTRIDE
    nblk = jnp.where(sid == NT - 1, (N - ZSTRIDE * (NT - 1)) // ZB, ZSTRIDE // ZB)

    zv = jnp.zeros((VL,), jnp.float32)

    def zrow(i, carry):
        for q in range(DH // VL):
            zbuf[i, pl.ds(q * VL, VL)] = zv
        return carry

    lax.fori_loop(0, ZB, zrow, 0)

    def load_edges(c_hbm, r_hbm, v_hbm):
        base = sid * NCHP
        pltpu.sync_copy(c_hbm.at[pl.ds(base, NCHP)], col_v)
        pltpu.sync_copy(r_hbm.at[pl.ds(base, NCHP)], row_v)
        pltpu.sync_copy(v_hbm.at[pl.ds(base, NCHP)], val_v)

        def adj(i, carry):
            for q in range(CHUNK // VL):
                sl = pl.ds(q * VL, VL)
                col_v[i, sl] = col_v[i, sl] + coff
            return carry

        lax.fori_loop(0, NCHP, adj, 0)

    def half_pass(table_hbm, out_hbm):
        def zblk(i, carry):
            pltpu.sync_copy(zbuf, acc.at[pl.ds(zbase + i * ZB, ZB)])
            return carry

        lax.fori_loop(0, nblk, zblk, 0)
        plsc.subcore_barrier()

        def chunk_body(j, carry):
            pltpu.async_copy(table_hbm.at[col_v.at[j]], rows_v, sem).wait()

            def scale_g(g, c2):
                vv = val_v[j, pl.ds(g * VL, VL)]
                for lane in range(VL):
                    e = g * VL + lane
                    v = vv[lane]
                    for q in range(DH // VL):
                        sl = pl.ds(q * VL, VL)
                        rows_v[e, sl] = rows_v[e, sl] * v
                return c2

            lax.fori_loop(0, CHUNK // VL, scale_g, 0)
            pltpu.sync_copy(rows_v, acc.at[row_v.at[j]], add=True)
            return carry

        lax.fori_loop(0, NCHP, chunk_body, 0)
        plsc.subcore_barrier()

        def cblk(i, carry):
            rbase = zbase + i * ZB
            pltpu.sync_copy(acc.at[pl.ds(rbase, ZB)], bounce)
            pltpu.sync_copy(bounce, out_hbm.at[pl.ds(coff + rbase, ZB)])
            return carry

        lax.fori_loop(0, nblk, cblk, 0)
        plsc.subcore_barrier()

    def spmm(tables, outs):
        half_pass(tables[0], outs[0])
        half_pass(tables[1], outs[1])

    load_edges(c0, r0, v0)
    spmm((t_a, t_b), (y1a_a, y1a_b))
    spmm((y1a_a, y1a_b), (y2a_a, y2a_b))
    load_edges(c1, r1, v1)
    spmm((t_a, t_b), (y1b_a, y1b_b))
    spmm((y1b_a, y1b_b), (y2b_a, y2b_b))


def _tc_combine(ta, tb, a1a, a1b, a2a, a2b, b1a, b1b, b2a, b2b, w_ref, o_ref):
    w0 = w_ref[0]
    w1 = w_ref[1]
    w2 = w_ref[2]
    w3 = w_ref[3]
    w4 = w_ref[4]
    wm = w0 - w2 - w4
    dot = functools.partial(jnp.dot, preferred_element_type=jnp.float32)
    o_ref[0] = (dot(ta[0], wm[:DH]) + dot(tb[0], wm[DH:])
                + dot(a1a[0], w1[:DH]) + dot(a1b[0], w1[DH:])
                + dot(a2a[0], 2.0 * w2[:DH]) + dot(a2b[0], 2.0 * w2[DH:])
                + dot(b1a[0], w3[:DH]) + dot(b1b[0], w3[DH:])
                + dot(b2a[0], 2.0 * w4[:DH]) + dot(b2b[0], 2.0 * w4[DH:]))


def _pad_tiles(a):
    a = a.reshape(NT, NCH, CHUNK)
    return jnp.pad(a, ((0, 0), (0, NCHP - NCH), (0, 0))).reshape(NT * NCHP, CHUNK)


def kernel(A0_indices, A0_values, A1_indices, A1_values, X, W):
    B, d, n = X.shape
    out_f = W.shape[1]

    T = jnp.transpose(X, (0, 2, 1)).reshape(B * n, d)
    t_a = T[:, :DH]
    t_b = T[:, DH:]

    def prep_idx(a):
        return _pad_tiles(a.astype(jnp.int32).reshape(E // CHUNK, CHUNK))

    c0 = prep_idx(A0_indices[1])
    r0 = prep_idx(A0_indices[0])
    v0 = _pad_tiles(A0_values.reshape(E // CHUNK, CHUNK))
    c1 = prep_idx(A1_indices[1])
    r1 = prep_idx(A1_indices[0])
    v1 = _pad_tiles(A1_values.reshape(E // CHUNK, CHUNK))

    sck = pl.kernel(
        _sc_body,
        out_type=[jax.ShapeDtypeStruct((B * n, DH), jnp.float32)] * 8,
        mesh=plsc.VectorSubcoreMesh(core_axis_name="c", subcore_axis_name="s"),
        compiler_params=pltpu.CompilerParams(use_tc_tiling_on_sc=False),
        scratch_types=[
            pltpu.VMEM((NCHP, CHUNK), jnp.int32),
            pltpu.VMEM((NCHP, CHUNK), jnp.int32),
            pltpu.VMEM((NCHP, CHUNK), jnp.float32),
            pltpu.VMEM((CHUNK, DH), jnp.float32),
            pltpu.VMEM((ZB, DH), jnp.float32),
            pltpu.VMEM((ZB, DH), jnp.float32),
            pltpu.VMEM_SHARED((N, DH), jnp.float32),
            pltpu.SemaphoreType.DMA,
        ],
    )
    ys = sck(t_a, t_b, c0, r0, v0, c1, r1, v1)

    Wk = W.reshape(d, 5, out_f).transpose(1, 0, 2)

    BN = 1000
    feats = [t_a, t_b] + list(ys)
    feats = [f.reshape(B, n, DH) for f in feats]
    fspec = pl.BlockSpec((1, BN, DH), lambda b, i: (b, i, 0))
    out = pl.pallas_call(
        _tc_combine,
        grid=(B, n // BN),
        in_specs=[fspec] * 10 + [pl.BlockSpec((5, d, out_f), lambda b, i: (0, 0, 0))],
        out_specs=pl.BlockSpec((1, BN, out_f), lambda b, i: (b, i, 0)),
        out_shape=jax.ShapeDtypeStruct((B, n, out_f), jnp.float32),
    )(*feats, Wk)
    return out


# speedup vs baseline: 2.6541x; 1.7855x over previous
import functools

import jax
import jax.numpy as jnp
from jax import lax
from jax.experimental import pallas as pl
from jax.experimental.pallas import tpu as pltpu
from jax.experimental.pallas import tpu_sc as plsc

N = 10000
E = 160000
D = 128
DH = 64
NT = 16
CHUNK = 80
NCH = E // NT // CHUNK
NCHP = 128
ZB = 80
ZSTRIDE = 640
VL = 16


def _sc_body(t_a, t_b, c0, r0, v0, c1, r1, v1,
             y1a_a, y1a_b, y2a_a, y2a_b, y1b_a, y1b_b, y2b_a, y2b_b,
             col_v, row_v, val_v, rows_v, zbuf, bounce, acc, sem_g, sem_s):
    cid = lax.axis_index("c")
    sid = lax.axis_index("s")
    coff = cid * N
    zbase = sid * ZSTRIDE
    nblk = jnp.where(sid == NT - 1, (N - ZSTRIDE * (NT - 1)) // ZB, ZSTRIDE // ZB)

    zv = jnp.zeros((VL,), jnp.float32)

    def zrow(i, carry):
        for q in range(DH // VL):
            zbuf[i, pl.ds(q * VL, VL)] = zv
        return carry

    lax.fori_loop(0, ZB, zrow, 0)

    def load_edges(c_hbm, r_hbm, v_hbm):
        base = sid * NCHP
        pltpu.sync_copy(c_hbm.at[pl.ds(base, NCHP)], col_v)
        pltpu.sync_copy(r_hbm.at[pl.ds(base, NCHP)], row_v)
        pltpu.sync_copy(v_hbm.at[pl.ds(base, NCHP)], val_v)

        def adj(i, carry):
            for q in range(CHUNK // VL):
                sl = pl.ds(q * VL, VL)
                col_v[i, sl] = col_v[i, sl] + coff
            return carry

        lax.fori_loop(0, NCHP, adj, 0)

    def half_pass(table_hbm, out_hbm):
        def zblk(i, carry):
            pltpu.sync_copy(zbuf, acc.at[pl.ds(zbase + i * ZB, ZB)])
            return carry

        lax.fori_loop(0, nblk, zblk, 0)
        plsc.subcore_barrier()

        def gslot(b):
            return rows_v.at[pl.ds(b * CHUNK, CHUNK)]

        pltpu.async_copy(table_hbm.at[col_v.at[0]], gslot(0), sem_g)
        pltpu.async_copy(table_hbm.at[col_v.at[1]], gslot(1), sem_g)

        def chunk_body(j, carry):
            b = lax.rem(j, 4)
            bn = lax.rem(j + 2, 4)
            pltpu.make_async_copy(table_hbm.at[col_v.at[j]], gslot(b), sem_g).wait()

            @pl.when(j >= 2)
            def _():
                pltpu.make_async_copy(gslot(bn), acc.at[row_v.at[j]], sem_s).wait()

            @pl.when(j < NCHP - 2)
            def _():
                pltpu.async_copy(table_hbm.at[col_v.at[j + 2]], gslot(bn), sem_g)

            be = b * CHUNK

            def scale_g(g, c2):
                vv = val_v[j, pl.ds(g * VL, VL)]
                for lane in range(VL):
                    e = be + g * VL + lane
                    v = vv[lane]
                    for q in range(DH // VL):
                        sl = pl.ds(q * VL, VL)
                        rows_v[e, sl] = rows_v[e, sl] * v
                return c2

            lax.fori_loop(0, CHUNK // VL, scale_g, 0)
            pltpu.async_copy(gslot(b), acc.at[row_v.at[j]], sem_s, add=True)
            return carry

        lax.fori_loop(0, NCHP, chunk_body, 0)
        pltpu.make_async_copy(gslot(2), acc.at[row_v.at[0]], sem_s).wait()
        pltpu.make_async_copy(gslot(3), acc.at[row_v.at[0]], sem_s).wait()
        plsc.subcore_barrier()

        def cblk(i, carry):
            rbase = zbase + i * ZB
            pltpu.sync_copy(acc.at[pl.ds(rbase, ZB)], bounce)
            pltpu.sync_copy(bounce, out_hbm.at[pl.ds(coff + rbase, ZB)])
            return carry

        lax.fori_loop(0, nblk, cblk, 0)
        plsc.subcore_barrier()

    def spmm(tables, outs):
        half_pass(tables[0], outs[0])
        half_pass(tables[1], outs[1])

    load_edges(c0, r0, v0)
    spmm((t_a, t_b), (y1a_a, y1a_b))
    spmm((y1a_a, y1a_b), (y2a_a, y2a_b))
    load_edges(c1, r1, v1)
    spmm((t_a, t_b), (y1b_a, y1b_b))
    spmm((y1b_a, y1b_b), (y2b_a, y2b_b))


def _tc_combine(ta, tb, a1a, a1b, a2a, a2b, b1a, b1b, b2a, b2b, w_ref, o_ref):
    w0 = w_ref[0]
    w1 = w_ref[1]
    w2 = w_ref[2]
    w3 = w_ref[3]
    w4 = w_ref[4]
    wm = w0 - w2 - w4
    dot = functools.partial(jnp.dot, preferred_element_type=jnp.float32)
    o_ref[0] = (dot(ta[0], wm[:DH]) + dot(tb[0], wm[DH:])
                + dot(a1a[0], w1[:DH]) + dot(a1b[0], w1[DH:])
                + dot(a2a[0], 2.0 * w2[:DH]) + dot(a2b[0], 2.0 * w2[DH:])
                + dot(b1a[0], w3[:DH]) + dot(b1b[0], w3[DH:])
                + dot(b2a[0], 2.0 * w4[:DH]) + dot(b2b[0], 2.0 * w4[DH:]))


def _pad_tiles(a):
    a = a.reshape(NT, NCH, CHUNK)
    return jnp.pad(a, ((0, 0), (0, NCHP - NCH), (0, 0))).reshape(NT * NCHP, CHUNK)


def kernel(A0_indices, A0_values, A1_indices, A1_values, X, W):
    B, d, n = X.shape
    out_f = W.shape[1]

    T = jnp.transpose(X, (0, 2, 1)).reshape(B * n, d)
    t_a = T[:, :DH]
    t_b = T[:, DH:]

    def prep_idx(a):
        return _pad_tiles(a.astype(jnp.int32).reshape(E // CHUNK, CHUNK))

    c0 = prep_idx(A0_indices[1])
    r0 = prep_idx(A0_indices[0])
    v0 = _pad_tiles(A0_values.reshape(E // CHUNK, CHUNK))
    c1 = prep_idx(A1_indices[1])
    r1 = prep_idx(A1_indices[0])
    v1 = _pad_tiles(A1_values.reshape(E // CHUNK, CHUNK))

    sck = pl.kernel(
        _sc_body,
        out_type=[jax.ShapeDtypeStruct((B * n, DH), jnp.float32)] * 8,
        mesh=plsc.VectorSubcoreMesh(core_axis_name="c", subcore_axis_name="s"),
        compiler_params=pltpu.CompilerParams(use_tc_tiling_on_sc=False),
        scratch_types=[
            pltpu.VMEM((NCHP, CHUNK), jnp.int32),
            pltpu.VMEM((NCHP, CHUNK), jnp.int32),
            pltpu.VMEM((NCHP, CHUNK), jnp.float32),
            pltpu.VMEM((4 * CHUNK, DH), jnp.float32),
            pltpu.VMEM((ZB, DH), jnp.float32),
            pltpu.VMEM((ZB, DH), jnp.float32),
            pltpu.VMEM_SHARED((N, DH), jnp.float32),
            pltpu.SemaphoreType.DMA,
            pltpu.SemaphoreType.DMA,
        ],
    )
    ys = sck(t_a, t_b, c0, r0, v0, c1, r1, v1)

    Wk = W.reshape(d, 5, out_f).transpose(1, 0, 2)

    BN = 1000
    feats = [t_a, t_b] + list(ys)
    feats = [f.reshape(B, n, DH) for f in feats]
    fspec = pl.BlockSpec((1, BN, DH), lambda b, i: (b, i, 0))
    out = pl.pallas_call(
        _tc_combine,
        grid=(B, n // BN),
        in_specs=[fspec] * 10 + [pl.BlockSpec((5, d, out_f), lambda b, i: (0, 0, 0))],
        out_specs=pl.BlockSpec((1, BN, out_f), lambda b, i: (b, i, 0)),
        out_shape=jax.ShapeDtypeStruct((B, n, out_f), jnp.float32),
    )(*feats, Wk)
    return out
